# baseline (speedup 1.0000x reference)
import jax
import jax.numpy as jnp
from jax.experimental import pallas as pl
from jax.experimental.pallas import tpu as pltpu

ALPHA = 0.2
EPS = 1e-16
LOG2E = 1.4426950408889634


def _gat_block_kernel(h_ref, hblk_ref, adj_ref, w_ref, a_ref, out_ref):
    dout = w_ref.shape[1]
    wh0 = jnp.dot(h_ref[...], w_ref[...], preferred_element_type=jnp.float32)
    whb = jnp.dot(hblk_ref[...], w_ref[...], preferred_element_type=jnp.float32)
    s1l = jnp.dot(wh0, a_ref[:dout, :], preferred_element_type=jnp.float32) * LOG2E
    s1al = ALPHA * s1l
    s2b = (jnp.dot(whb, a_ref[dout:, :], preferred_element_type=jnp.float32) * LOG2E).T
    mask = adj_ref[...] > 0.0
    m1 = jnp.max(jnp.where(mask, s1l, -jnp.inf), axis=0, keepdims=True)
    mb = m1 + s2b
    m = jnp.maximum(mb, ALPHA * mb)
    c1 = s2b - m
    c2 = ALPHA * s2b - m
    t = jnp.maximum(s1l + c1, s1al + c2)
    p = jnp.exp2(jnp.where(mask, t, -jnp.inf))
    denom = jnp.sum(p, axis=0, keepdims=True) + EPS
    hp = jax.lax.dot_general(
        p.astype(jnp.bfloat16), wh0.astype(jnp.bfloat16),
        (((0,), (0,)), ((), ())),
        preferred_element_type=jnp.float32,
    ) * (1.0 / denom).T
    out_ref[...] = jnp.where(hp > 0.0, hp, jnp.exp(hp) - 1.0)


def kernel(h, adj, W, a):
    N, din = h.shape
    dout = W.shape[1]
    blk = 512
    grid = N // blk
    return pl.pallas_call(
        _gat_block_kernel,
        grid=(grid,),
        in_specs=[
            pl.BlockSpec((N, din), lambda i: (0, 0)),
            pl.BlockSpec((blk, din), lambda i: (i, 0)),
            pl.BlockSpec((N, blk), lambda i: (0, i)),
            pl.BlockSpec((din, dout), lambda i: (0, 0)),
            pl.BlockSpec((2 * dout, 1), lambda i: (0, 0)),
        ],
        out_specs=pl.BlockSpec((blk, dout), lambda i: (i, 0)),
        out_shape=jax.ShapeDtypeStruct((N, dout), jnp.float32),
        compiler_params=pltpu.CompilerParams(
            dimension_semantics=("parallel",),
        ),
    )(h, h, adj, W, a)


# speedup vs baseline: 1.0693x; 1.0693x over previous
import jax
import jax.numpy as jnp
from jax.experimental import pallas as pl
from jax.experimental.pallas import tpu as pltpu

ALPHA = 0.2
EPS = 1e-16
LOG2E = 1.4426950408889634
BLK = 512


def _process(adj_blk, s1l, s1al, s2b, wh16):
    mask = adj_blk > 0.0
    m1 = jnp.max(jnp.where(mask, s1l, -jnp.inf), axis=0, keepdims=True)
    mb = m1 + s2b
    m = jnp.maximum(mb, ALPHA * mb)
    c1 = s2b - m
    c2 = ALPHA * s2b - m
    t = jnp.maximum(s1l + c1, s1al + c2)
    p = jnp.exp2(jnp.where(mask, t, -jnp.inf))
    denom = jnp.sum(p, axis=0, keepdims=True) + EPS
    hp = jax.lax.dot_general(
        p.astype(jnp.bfloat16), wh16,
        (((0,), (0,)), ((), ())),
        preferred_element_type=jnp.float32,
    ) * (1.0 / denom).T
    return jnp.where(hp > 0.0, hp, jnp.exp(hp) - 1.0)


def _gat_kernel(h_ref, adj_ref, w_ref, a_ref, out_ref, buf_ref, sem):
    dout = w_ref.shape[1]
    c0 = pltpu.make_async_copy(adj_ref.at[:, 0:BLK], buf_ref.at[0], sem.at[0])
    c1 = pltpu.make_async_copy(adj_ref.at[:, BLK:2 * BLK], buf_ref.at[1],
                               sem.at[1])
    c0.start()
    c1.start()
    wh0 = jnp.dot(h_ref[...], w_ref[...], preferred_element_type=jnp.float32)
    wh16 = wh0.astype(jnp.bfloat16)
    s1l = jnp.dot(wh0, a_ref[:dout, :],
                  preferred_element_type=jnp.float32) * LOG2E
    s1al = ALPHA * s1l
    s2t = (jnp.dot(wh0, a_ref[dout:, :],
                   preferred_element_type=jnp.float32) * LOG2E).T
    c0.wait()
    out_ref[0:BLK, :] = _process(buf_ref[0], s1l, s1al, s2t[:, 0:BLK], wh16)
    c1.wait()
    out_ref[BLK:2 * BLK, :] = _process(buf_ref[1], s1l, s1al,
                                       s2t[:, BLK:2 * BLK], wh16)


def kernel(h, adj, W, a):
    N, din = h.shape
    dout = W.shape[1]
    return pl.pallas_call(
        _gat_kernel,
        in_specs=[
            pl.BlockSpec(memory_space=pltpu.VMEM),
            pl.BlockSpec(memory_space=pl.ANY),
            pl.BlockSpec(memory_space=pltpu.VMEM),
            pl.BlockSpec(memory_space=pltpu.VMEM),
        ],
        out_specs=pl.BlockSpec(memory_space=pltpu.VMEM),
        out_shape=jax.ShapeDtypeStruct((N, dout), jnp.float32),
        scratch_shapes=[
            pltpu.VMEM((2, N, BLK), jnp.float32),
            pltpu.SemaphoreType.DMA((2,)),
        ],
    )(h, adj, W, a)


# speedup vs baseline: 1.1765x; 1.1002x over previous
import jax
import jax.numpy as jnp
from jax.experimental import pallas as pl
from jax.experimental.pallas import tpu as pltpu

ALPHA = 0.2
EPS = 1e-16
LOG2E = 1.4426950408889634


def _gat_block_kernel(h_ref, adj_ref, w_ref, a_ref, out_ref,
                      wh_ref, s1l_ref, s1al_ref, s2t_ref):
    dout = w_ref.shape[1]
    blk = adj_ref.shape[1]

    @pl.when(pl.program_id(0) == 0)
    def _():
        wh0 = jnp.dot(h_ref[...], w_ref[...], preferred_element_type=jnp.float32)
        wh_ref[...] = wh0.astype(jnp.bfloat16)
        s1 = jnp.dot(wh0, a_ref[:dout, :],
                     preferred_element_type=jnp.float32) * LOG2E
        s1l_ref[...] = s1
        s1al_ref[...] = ALPHA * s1
        s2t_ref[...] = (jnp.dot(wh0, a_ref[dout:, :],
                                preferred_element_type=jnp.float32) * LOG2E).T

    j0 = pl.program_id(0) * blk
    s1l = s1l_ref[...]
    s1al = s1al_ref[...]
    s2b = s2t_ref[:, pl.ds(j0, blk)]
    mask = adj_ref[...] > 0.0
    m1 = jnp.max(jnp.where(mask, s1l, -jnp.inf), axis=0, keepdims=True)
    mb = m1 + s2b
    m = jnp.maximum(mb, ALPHA * mb)
    c1 = s2b - m
    c2 = ALPHA * s2b - m
    t = jnp.maximum(s1l + c1, s1al + c2)
    p = jnp.exp2(jnp.where(mask, t, -jnp.inf))
    denom = jnp.sum(p, axis=0, keepdims=True) + EPS
    hp = jax.lax.dot_general(
        p.astype(jnp.bfloat16), wh_ref[...],
        (((0,), (0,)), ((), ())),
        preferred_element_type=jnp.float32,
    ) * (1.0 / denom).T
    out_ref[...] = jnp.where(hp > 0.0, hp, jnp.exp(hp) - 1.0)


def kernel(h, adj, W, a):
    N, din = h.shape
    dout = W.shape[1]
    blk = 512
    grid = N // blk
    return pl.pallas_call(
        _gat_block_kernel,
        grid=(grid,),
        in_specs=[
            pl.BlockSpec((N, din), lambda i: (0, 0)),
            pl.BlockSpec((N, blk), lambda i: (0, i)),
            pl.BlockSpec((din, dout), lambda i: (0, 0)),
            pl.BlockSpec((2 * dout, 1), lambda i: (0, 0)),
        ],
        out_specs=pl.BlockSpec((blk, dout), lambda i: (i, 0)),
        out_shape=jax.ShapeDtypeStruct((N, dout), jnp.float32),
        scratch_shapes=[
            pltpu.VMEM((N, dout), jnp.bfloat16),
            pltpu.VMEM((N, 1), jnp.float32),
            pltpu.VMEM((N, 1), jnp.float32),
            pltpu.VMEM((1, N), jnp.float32),
        ],
    )(h, adj, W, a)


# speedup vs baseline: 1.1974x; 1.0178x over previous
import jax
import jax.numpy as jnp
from jax.experimental import pallas as pl
from jax.experimental.pallas import tpu as pltpu

ALPHA = 0.2
EPS = 1e-16
LOG2E = 1.4426950408889634


def _gat_block_kernel(h_ref, adj_ref, w_ref, a_ref, out_ref,
                      wh_ref, s1l_ref, s1al_ref, s2t_ref):
    dout = w_ref.shape[1]
    blk = adj_ref.shape[1]

    @pl.when(pl.program_id(0) == 0)
    def _():
        wh0 = jnp.dot(h_ref[...], w_ref[...], preferred_element_type=jnp.float32)
        wh_ref[...] = jnp.concatenate(
            [wh0, jnp.ones_like(wh0[:, :1]), jnp.zeros_like(wh0[:, 1:])],
            axis=1).astype(jnp.bfloat16)
        s1 = jnp.dot(wh0, a_ref[:dout, :],
                     preferred_element_type=jnp.float32) * LOG2E
        s1l_ref[...] = s1
        s1al_ref[...] = ALPHA * s1
        s2t_ref[...] = (jnp.dot(wh0, a_ref[dout:, :],
                                preferred_element_type=jnp.float32) * LOG2E).T

    j0 = pl.program_id(0) * blk
    s1l = s1l_ref[...]
    s1al = s1al_ref[...]
    s2b = s2t_ref[:, pl.ds(j0, blk)]
    mask = adj_ref[...] > 0.0
    m1 = jnp.max(jnp.where(mask, s1l, -jnp.inf), axis=0, keepdims=True)
    mb = m1 + s2b
    m = jnp.maximum(mb, ALPHA * mb)
    c1 = s2b - m
    c2 = ALPHA * s2b - m
    t = jnp.maximum(s1l + c1, s1al + c2)
    p = jnp.exp2(jnp.where(mask, t, -jnp.inf))
    hp_aug = jax.lax.dot_general(
        p.astype(jnp.bfloat16), wh_ref[...],
        (((0,), (0,)), ((), ())),
        preferred_element_type=jnp.float32,
    )
    denom = hp_aug[:, dout:dout + 1] + EPS
    hp = hp_aug[:, :dout] * (1.0 / denom)
    out_ref[...] = jnp.where(hp > 0.0, hp, jnp.exp(hp) - 1.0)


def kernel(h, adj, W, a):
    N, din = h.shape
    dout = W.shape[1]
    blk = 512
    grid = N // blk
    return pl.pallas_call(
        _gat_block_kernel,
        grid=(grid,),
        in_specs=[
            pl.BlockSpec((N, din), lambda i: (0, 0)),
            pl.BlockSpec((N, blk), lambda i: (0, i)),
            pl.BlockSpec((din, dout), lambda i: (0, 0)),
            pl.BlockSpec((2 * dout, 1), lambda i: (0, 0)),
        ],
        out_specs=pl.BlockSpec((blk, dout), lambda i: (i, 0)),
        out_shape=jax.ShapeDtypeStruct((N, dout), jnp.float32),
        scratch_shapes=[
            pltpu.VMEM((N, 2 * dout), jnp.bfloat16),
            pltpu.VMEM((N, 1), jnp.float32),
            pltpu.VMEM((N, 1), jnp.float32),
            pltpu.VMEM((1, N), jnp.float32),
        ],
    )(h, adj, W, a)
